# baseline (speedup 1.0000x reference)
import functools

import jax
import jax.numpy as jnp
from jax import lax
from jax.experimental import pallas as pl
from jax.experimental.pallas import tpu as pltpu
from jax.experimental.pallas import tpu_sc as plsc

N = 4096
E = 16384
B = 16
D = 300
DP = 384
DEPTH = 3
NC = 2
NS = 16
CHUNK = 1024
NCH = E // CHUNK
PP = 16
RPT = N // NS
LANES = 16


def _pad2(w, rows, cols):
    return jnp.zeros((rows, cols), w.dtype).at[: w.shape[0], : w.shape[1]].set(w)


def _linear(x, w, b, relu, block_rows):
    m, k = x.shape
    dp = w.shape[1]

    def body(x_ref, w_ref, b_ref, o_ref):
        y = jnp.dot(x_ref[...], w_ref[...],
                    preferred_element_type=jnp.float32) + b_ref[...]
        o_ref[...] = jnp.maximum(y, 0.0) if relu else y

    return pl.pallas_call(
        body,
        grid=(m // block_rows,),
        in_specs=[
            pl.BlockSpec((block_rows, k), lambda i: (i, 0)),
            pl.BlockSpec((k, dp), lambda i: (0, 0)),
            pl.BlockSpec((1, dp), lambda i: (0, 0)),
        ],
        out_specs=pl.BlockSpec((block_rows, dp), lambda i: (i, 0)),
        out_shape=jax.ShapeDtypeStruct((m, dp), jnp.float32),
    )(x, w, b)


def _mlp(h, agg, wa, ba, wb, bb, relu, block_rows=512):
    rows = h.shape[0]

    def body(h_ref, a_ref, wa_ref, ba_ref, wb_ref, bb_ref, o_ref):
        z = h_ref[...] + a_ref[...]
        t = jnp.maximum(
            jnp.dot(z, wa_ref[...], preferred_element_type=jnp.float32)
            + ba_ref[...], 0.0)
        y = jnp.dot(t, wb_ref[...],
                    preferred_element_type=jnp.float32) + bb_ref[...]
        o_ref[...] = jnp.maximum(y, 0.0) if relu else y

    return pl.pallas_call(
        body,
        grid=(rows // block_rows,),
        in_specs=[
            pl.BlockSpec((block_rows, DP), lambda i: (i, 0)),
            pl.BlockSpec((block_rows, DP), lambda i: (i, 0)),
            pl.BlockSpec((DP, DP), lambda i: (0, 0)),
            pl.BlockSpec((1, DP), lambda i: (0, 0)),
            pl.BlockSpec((DP, DP), lambda i: (0, 0)),
            pl.BlockSpec((1, DP), lambda i: (0, 0)),
        ],
        out_specs=pl.BlockSpec((block_rows, DP), lambda i: (i, 0)),
        out_shape=jax.ShapeDtypeStruct((rows, DP), jnp.float32),
    )(h, agg, wa, ba, wb, bb)


def _final(h, agg, wa, ba, wb, bb, seg3, block_rows=512):
    nblk = 2 * N // block_rows
    half = N // block_rows

    def body(h_ref, a_ref, wa_ref, ba_ref, wb_ref, bb_ref, sg,
             diff_ref, re_ref, pr_ref):
        j = pl.program_id(0)
        z = h_ref[...] + a_ref[...]
        t = jnp.maximum(
            jnp.dot(z, wa_ref[...], preferred_element_type=jnp.float32)
            + ba_ref[...], 0.0)
        o = jnp.dot(t, wb_ref[...],
                    preferred_element_type=jnp.float32) + bb_ref[...]
        iota = lax.broadcasted_iota(jnp.int32, (B, block_rows), 0)
        ct = jnp.dot((sg[0] == iota).astype(jnp.float32), o,
                     preferred_element_type=jnp.float32)

        @pl.when(j == 0)
        def _():
            re_ref[...] = ct

        @pl.when((j > 0) & (j < half))
        def _():
            re_ref[...] += ct

        @pl.when(j == half)
        def _():
            pr_ref[...] = ct

        @pl.when(j > half)
        def _():
            pr_ref[...] += ct

        @pl.when(j == nblk - 1)
        def _():
            diff_ref[...] = re_ref[...] - pr_ref[...]

    row_spec = pl.BlockSpec((block_rows, DP), lambda i: (i, 0))
    full = pl.BlockSpec((DP, DP), lambda i: (0, 0))
    bias = pl.BlockSpec((1, DP), lambda i: (0, 0))
    seg_spec = pl.BlockSpec((1, 1, block_rows), lambda i: (i, 0, 0))
    out_spec = pl.BlockSpec((B, DP), lambda i: (0, 0))
    return pl.pallas_call(
        body,
        grid=(nblk,),
        in_specs=[row_spec, row_spec, full, bias, full, bias, seg_spec],
        out_specs=(out_spec, out_spec, out_spec),
        out_shape=(jax.ShapeDtypeStruct((B, DP), jnp.float32),
                   jax.ShapeDtypeStruct((B, DP), jnp.float32),
                   jax.ShapeDtypeStruct((B, DP), jnp.float32)),
    )(h, agg, wa, ba, wb, bb, seg3)


def _edge_agg(h, e, src, dst):
    mesh = plsc.VectorSubcoreMesh(core_axis_name="c", subcore_axis_name="s",
                                  num_cores=NC, num_subcores=NS)

    @functools.partial(
        pl.kernel,
        out_type=jax.ShapeDtypeStruct((2 * N, DP), jnp.float32),
        mesh=mesh,
        compiler_params=pltpu.CompilerParams(needs_layout_passes=False),
        scratch_types=[
            pltpu.VMEM((CHUNK,), jnp.int32),
            pltpu.VMEM((CHUNK,), jnp.int32),
            pltpu.VMEM((CHUNK + PP,), jnp.int32),
            pltpu.VMEM((CHUNK + PP,), jnp.int32),
            pltpu.VMEM((CHUNK + PP,), jnp.int32),
            pltpu.VMEM((2 * PP, DP), jnp.float32),
            pltpu.VMEM((2 * PP, DP), jnp.float32),
            pltpu.VMEM((RPT, DP), jnp.float32),
            pltpu.SemaphoreType.DMA,
            pltpu.SemaphoreType.DMA,
            pltpu.SemaphoreType.DMA,
            pltpu.SemaphoreType.DMA,
        ],
    )
    def k(hh, ee, ss, dd, out,
          dbuf, sbuf, plist, slist, rlist, hbuf, ebuf, acc,
          semd, semsrc, semb0, semb1):
        cid = lax.axis_index("c")
        sid = lax.axis_index("s")
        row0 = sid * RPT
        ebase = cid * E
        hoff0 = cid * N
        zero16 = jnp.zeros((LANES,), jnp.float32)
        zero16i = jnp.zeros((LANES,), jnp.int32)

        def zacc(r, carry):
            for j in range(DP // LANES):
                acc[r, pl.ds(j * LANES, LANES)] = zero16
            return carry

        lax.fori_loop(0, RPT, zacc, 0)

        def zidx(v, carry):
            plist[pl.ds(v * LANES, LANES)] = zero16i
            slist[pl.ds(v * LANES, LANES)] = zero16i
            return carry

        lax.fori_loop(0, (CHUNK + PP) // LANES, zidx, 0)

        pltpu.async_copy(dd.at[pl.ds(ebase, CHUNK)], dbuf, semd)
        pltpu.async_copy(ss.at[pl.ds(ebase, CHUNK)], sbuf, semsrc)

        def issue(off, half, sb):
            pltpu.async_copy(hh.at[slist.at[pl.ds(off, PP)]],
                             hbuf.at[pl.ds(half * PP, PP)], sb)
            pltpu.async_copy(ee.at[plist.at[pl.ds(off, PP)]],
                             ebuf.at[pl.ds(half * PP, PP)], sb)

        def drain(sb):
            pltpu.make_async_copy(hh.at[pl.ds(0, 2 * PP)], hbuf, sb).wait()

        def chunk_body(kk, carry):
            base = ebase + kk * CHUNK
            pltpu.make_async_copy(dd.at[pl.ds(base, CHUNK)], dbuf,
                                  semd).wait()
            pltpu.make_async_copy(ss.at[pl.ds(base, CHUNK)], sbuf,
                                  semsrc).wait()

            def scan(v, ncur):
                sl = pl.ds(v * LANES, LANES)
                ids = dbuf[sl]
                srcs = sbuf[sl]
                mask = (ids >= row0) & (ids < row0 + RPT)
                cnt = plsc.all_reduce_population_count(mask)

                @pl.when(cnt[0] > 0)
                def _():
                    iota = lax.broadcasted_iota(jnp.int32, (LANES,), 0)
                    pos = iota + (base + v * LANES)
                    mi = mask.astype(jnp.int32)
                    pref = mi
                    for kshift in (1, 2, 4, 8):
                        shifted = jnp.take_along_axis(
                            pref, jnp.maximum(iota - kshift, 0), axis=0)
                        pref = pref + jnp.where(iota >= kshift, shifted, 0)
                    idxv = pref - mi + ncur
                    plsc.store_scatter(plist, [idxv], pos, mask=mask)
                    plsc.store_scatter(slist, [idxv], srcs + hoff0, mask=mask)
                    plsc.store_scatter(rlist, [idxv], ids - row0, mask=mask)

                return ncur + cnt[0]

            n = lax.fori_loop(0, CHUNK // LANES, scan, jnp.int32(0))

            @pl.when(kk + 1 < NCH)
            def _():
                pltpu.async_copy(dd.at[pl.ds(base + CHUNK, CHUNK)],
                                 dbuf, semd)
                pltpu.async_copy(ss.at[pl.ds(base + CHUNK, CHUNK)],
                                 sbuf, semsrc)

            @pl.when(n > 0)
            def _():
                issue(0, 0, semb0)

            def gb_body(b, carry2):
                off = b * PP
                par = b % 2

                @pl.when(par == 0)
                def _():
                    drain(semb0)

                @pl.when(par == 1)
                def _():
                    drain(semb1)

                @pl.when(off + PP < n)
                def _():
                    @pl.when(par == 0)
                    def _():
                        issue(off + PP, 1, semb1)

                    @pl.when(par == 1)
                    def _():
                        issue(off + PP, 0, semb0)

                hoff = par * PP

                def rowacc(r, carry3):
                    lrow = rlist[pl.ds(off + r, LANES)][0]
                    hrow = hoff + r
                    for j in range(DP // LANES):
                        sl2 = pl.ds(j * LANES, LANES)
                        plsc.addupdate(acc.at[lrow, sl2], jnp.maximum(
                            hbuf[hrow, sl2] + ebuf[hrow, sl2], 0.0))
                    return carry3

                lax.fori_loop(0, jnp.minimum(PP, n - off), rowacc, 0)
                return carry2

            lax.fori_loop(0, (n + PP - 1) // PP, gb_body, 0)
            return carry

        lax.fori_loop(0, NCH, chunk_body, 0)
        pltpu.sync_copy(acc, out.at[pl.ds(hoff0 + row0, RPT)])

    return k(h, e, src, dst)


def kernel(node_feats_r, edge_feats_r, node_feats_p, edge_feats_p,
           Wn, bn, We, be, Wa, ba, Wb, bb,
           edge_index_r, seg_r, edge_index_p, seg_p):
    f32 = jnp.float32
    wn = _pad2(Wn, 64, DP)
    we = _pad2(We, 8, DP)
    bn2 = _pad2(bn[None, :], 1, DP)
    be2 = _pad2(be[None, :], 1, DP)
    wa = [_pad2(Wa[i], DP, DP) for i in range(DEPTH)]
    wb = [_pad2(Wb[i], DP, DP) for i in range(DEPTH)]
    ba2 = [_pad2(ba[i][None, :], 1, DP) for i in range(DEPTH)]
    bb2 = [_pad2(bb[i][None, :], 1, DP) for i in range(DEPTH)]

    src = jnp.concatenate([edge_index_r[0], edge_index_p[0]]).astype(jnp.int32)
    dst = jnp.concatenate([edge_index_r[1], edge_index_p[1]]).astype(jnp.int32)
    seg3 = jnp.concatenate([seg_r, seg_p]).astype(jnp.int32).reshape(
        2 * N // 512, 1, 512)

    nf = jnp.concatenate([node_feats_r, node_feats_p]).astype(f32)
    ef = jnp.concatenate([edge_feats_r, edge_feats_p]).astype(f32)

    h = _linear(nf, wn, bn2, True, 512)
    e = _linear(ef, we, be2, False, 2048)

    for i in range(DEPTH - 1):
        agg = _edge_agg(h, e, src, dst)
        h = _mlp(h, agg, wa[i], ba2[i], wb[i], bb2[i], True)

    agg = _edge_agg(h, e, src, dst)
    diff, react, prod = _final(h, agg, wa[2], ba2[2], wb[2], bb2[2], seg3)
    return (diff[:, :D], react[:, :D], prod[:, :D])


# speedup vs baseline: 1.0568x; 1.0568x over previous
import functools

import jax
import jax.numpy as jnp
from jax import lax
from jax.experimental import pallas as pl
from jax.experimental.pallas import tpu as pltpu
from jax.experimental.pallas import tpu_sc as plsc

N = 4096
E = 16384
B = 16
D = 300
DP = 384
DEPTH = 3
NC = 2
NS = 16
CHUNK = 1024
NCH = E // CHUNK
PP = 16
RPT = N // NS
LANES = 16


def _pad2(w, rows, cols):
    return jnp.zeros((rows, cols), w.dtype).at[: w.shape[0], : w.shape[1]].set(w)


def _linear(x, w, b, relu, block_rows):
    m, k = x.shape
    dp = w.shape[1]

    def body(x_ref, w_ref, b_ref, o_ref):
        y = jnp.dot(x_ref[...], w_ref[...],
                    preferred_element_type=jnp.float32) + b_ref[...]
        o_ref[...] = jnp.maximum(y, 0.0) if relu else y

    return pl.pallas_call(
        body,
        grid=(m // block_rows,),
        in_specs=[
            pl.BlockSpec((block_rows, k), lambda i: (i, 0)),
            pl.BlockSpec((k, dp), lambda i: (0, 0)),
            pl.BlockSpec((1, dp), lambda i: (0, 0)),
        ],
        out_specs=pl.BlockSpec((block_rows, dp), lambda i: (i, 0)),
        out_shape=jax.ShapeDtypeStruct((m, dp), jnp.float32),
    )(x, w, b)


def _mlp(h, agg, wa, ba, wb, bb, relu, block_rows=512):
    rows = h.shape[0]

    def body(h_ref, a_ref, wa_ref, ba_ref, wb_ref, bb_ref, o_ref):
        z = h_ref[...] + a_ref[...]
        t = jnp.maximum(
            jnp.dot(z, wa_ref[...], preferred_element_type=jnp.float32)
            + ba_ref[...], 0.0)
        y = jnp.dot(t, wb_ref[...],
                    preferred_element_type=jnp.float32) + bb_ref[...]
        o_ref[...] = jnp.maximum(y, 0.0) if relu else y

    return pl.pallas_call(
        body,
        grid=(rows // block_rows,),
        in_specs=[
            pl.BlockSpec((block_rows, DP), lambda i: (i, 0)),
            pl.BlockSpec((block_rows, DP), lambda i: (i, 0)),
            pl.BlockSpec((DP, DP), lambda i: (0, 0)),
            pl.BlockSpec((1, DP), lambda i: (0, 0)),
            pl.BlockSpec((DP, DP), lambda i: (0, 0)),
            pl.BlockSpec((1, DP), lambda i: (0, 0)),
        ],
        out_specs=pl.BlockSpec((block_rows, DP), lambda i: (i, 0)),
        out_shape=jax.ShapeDtypeStruct((rows, DP), jnp.float32),
    )(h, agg, wa, ba, wb, bb)


def _final(h, agg, wa, ba, wb, bb, seg3, block_rows=512):
    nblk = 2 * N // block_rows
    half = N // block_rows

    def body(h_ref, a_ref, wa_ref, ba_ref, wb_ref, bb_ref, sg,
             diff_ref, re_ref, pr_ref):
        j = pl.program_id(0)
        z = h_ref[...] + a_ref[...]
        t = jnp.maximum(
            jnp.dot(z, wa_ref[...], preferred_element_type=jnp.float32)
            + ba_ref[...], 0.0)
        o = jnp.dot(t, wb_ref[...],
                    preferred_element_type=jnp.float32) + bb_ref[...]
        iota = lax.broadcasted_iota(jnp.int32, (B, block_rows), 0)
        ct = jnp.dot((sg[0] == iota).astype(jnp.float32), o,
                     preferred_element_type=jnp.float32)

        @pl.when(j == 0)
        def _():
            re_ref[...] = ct

        @pl.when((j > 0) & (j < half))
        def _():
            re_ref[...] += ct

        @pl.when(j == half)
        def _():
            pr_ref[...] = ct

        @pl.when(j > half)
        def _():
            pr_ref[...] += ct

        @pl.when(j == nblk - 1)
        def _():
            diff_ref[...] = re_ref[...] - pr_ref[...]

    row_spec = pl.BlockSpec((block_rows, DP), lambda i: (i, 0))
    full = pl.BlockSpec((DP, DP), lambda i: (0, 0))
    bias = pl.BlockSpec((1, DP), lambda i: (0, 0))
    seg_spec = pl.BlockSpec((1, 1, block_rows), lambda i: (i, 0, 0))
    out_spec = pl.BlockSpec((B, DP), lambda i: (0, 0))
    return pl.pallas_call(
        body,
        grid=(nblk,),
        in_specs=[row_spec, row_spec, full, bias, full, bias, seg_spec],
        out_specs=(out_spec, out_spec, out_spec),
        out_shape=(jax.ShapeDtypeStruct((B, DP), jnp.float32),
                   jax.ShapeDtypeStruct((B, DP), jnp.float32),
                   jax.ShapeDtypeStruct((B, DP), jnp.float32)),
    )(h, agg, wa, ba, wb, bb, seg3)


def _edge_agg(h, e, src, dst):
    mesh = plsc.VectorSubcoreMesh(core_axis_name="c", subcore_axis_name="s",
                                  num_cores=NC, num_subcores=NS)

    @functools.partial(
        pl.kernel,
        out_type=jax.ShapeDtypeStruct((2 * N, DP), jnp.float32),
        mesh=mesh,
        compiler_params=pltpu.CompilerParams(needs_layout_passes=False),
        scratch_types=[
            pltpu.VMEM((CHUNK,), jnp.int32),
            pltpu.VMEM((CHUNK,), jnp.int32),
            pltpu.VMEM((CHUNK + PP,), jnp.int32),
            pltpu.VMEM((CHUNK + PP,), jnp.int32),
            pltpu.VMEM((CHUNK + PP,), jnp.int32),
            pltpu.VMEM((2 * PP, DP), jnp.float32),
            pltpu.VMEM((2 * PP, DP), jnp.float32),
            pltpu.VMEM((RPT, DP), jnp.float32),
            pltpu.SemaphoreType.DMA,
            pltpu.SemaphoreType.DMA,
            pltpu.SemaphoreType.DMA,
            pltpu.SemaphoreType.DMA,
        ],
    )
    def k(hh, ee, ss, dd, out,
          dbuf, sbuf, plist, slist, rlist, hbuf, ebuf, acc,
          semd, semsrc, semb0, semb1):
        cid = lax.axis_index("c")
        sid = lax.axis_index("s")
        row0 = sid * RPT
        ebase = cid * E
        hoff0 = cid * N
        zero16 = jnp.zeros((LANES,), jnp.float32)
        zero16i = jnp.zeros((LANES,), jnp.int32)

        def zacc(r, carry):
            for j in range(DP // LANES):
                acc[r, pl.ds(j * LANES, LANES)] = zero16
            return carry

        lax.fori_loop(0, RPT, zacc, 0)

        def zidx(v, carry):
            plist[pl.ds(v * LANES, LANES)] = zero16i
            slist[pl.ds(v * LANES, LANES)] = zero16i
            return carry

        lax.fori_loop(0, (CHUNK + PP) // LANES, zidx, 0)

        pltpu.async_copy(dd.at[pl.ds(ebase, CHUNK)], dbuf, semd)
        pltpu.async_copy(ss.at[pl.ds(ebase, CHUNK)], sbuf, semsrc)

        def issue(off, half, sb):
            pltpu.async_copy(hh.at[slist.at[pl.ds(off, PP)]],
                             hbuf.at[pl.ds(half * PP, PP)], sb)
            pltpu.async_copy(ee.at[plist.at[pl.ds(off, PP)]],
                             ebuf.at[pl.ds(half * PP, PP)], sb)

        def drain(sb):
            pltpu.make_async_copy(hh.at[pl.ds(0, 2 * PP)], hbuf, sb).wait()

        def chunk_body(kk, carry):
            base = ebase + kk * CHUNK
            pltpu.make_async_copy(dd.at[pl.ds(base, CHUNK)], dbuf,
                                  semd).wait()
            pltpu.make_async_copy(ss.at[pl.ds(base, CHUNK)], sbuf,
                                  semsrc).wait()

            def scan(v, ncur):
                sl = pl.ds(v * LANES, LANES)
                ids = dbuf[sl]
                srcs = sbuf[sl]
                mask = (ids >= row0) & (ids < row0 + RPT)
                iota = lax.broadcasted_iota(jnp.int32, (LANES,), 0)
                pos = iota + (base + v * LANES)
                mi = mask.astype(jnp.int32)
                pref = mi
                for kshift in (1, 2, 4, 8):
                    shifted = jnp.take_along_axis(
                        pref, jnp.maximum(iota - kshift, 0), axis=0)
                    pref = pref + jnp.where(iota >= kshift, shifted, 0)
                idxv = pref - mi + ncur
                plsc.store_scatter(plist, [idxv], pos, mask=mask)
                plsc.store_scatter(slist, [idxv], srcs + hoff0, mask=mask)
                plsc.store_scatter(rlist, [idxv], ids - row0, mask=mask)
                cnt = plsc.all_reduce_population_count(mask)
                return ncur + cnt[0]

            n = lax.fori_loop(0, CHUNK // LANES, scan, jnp.int32(0))

            @pl.when(kk + 1 < NCH)
            def _():
                pltpu.async_copy(dd.at[pl.ds(base + CHUNK, CHUNK)],
                                 dbuf, semd)
                pltpu.async_copy(ss.at[pl.ds(base + CHUNK, CHUNK)],
                                 sbuf, semsrc)

            @pl.when(n > 0)
            def _():
                issue(0, 0, semb0)

            def gb_body(b, carry2):
                off = b * PP
                par = b % 2

                @pl.when(par == 0)
                def _():
                    drain(semb0)

                @pl.when(par == 1)
                def _():
                    drain(semb1)

                @pl.when(off + PP < n)
                def _():
                    @pl.when(par == 0)
                    def _():
                        issue(off + PP, 1, semb1)

                    @pl.when(par == 1)
                    def _():
                        issue(off + PP, 0, semb0)

                hoff = par * PP

                def rowacc(r, carry3):
                    lrow = rlist[pl.ds(off + r, LANES)][0]
                    hrow = hoff + r
                    for j in range(DP // LANES):
                        sl2 = pl.ds(j * LANES, LANES)
                        plsc.addupdate(acc.at[lrow, sl2], jnp.maximum(
                            hbuf[hrow, sl2] + ebuf[hrow, sl2], 0.0))
                    return carry3

                lax.fori_loop(0, jnp.minimum(PP, n - off), rowacc, 0)
                return carry2

            lax.fori_loop(0, (n + PP - 1) // PP, gb_body, 0)
            return carry

        lax.fori_loop(0, NCH, chunk_body, 0)
        pltpu.sync_copy(acc, out.at[pl.ds(hoff0 + row0, RPT)])

    return k(h, e, src, dst)


def kernel(node_feats_r, edge_feats_r, node_feats_p, edge_feats_p,
           Wn, bn, We, be, Wa, ba, Wb, bb,
           edge_index_r, seg_r, edge_index_p, seg_p):
    f32 = jnp.float32
    wn = _pad2(Wn, 64, DP)
    we = _pad2(We, 8, DP)
    bn2 = _pad2(bn[None, :], 1, DP)
    be2 = _pad2(be[None, :], 1, DP)
    wa = [_pad2(Wa[i], DP, DP) for i in range(DEPTH)]
    wb = [_pad2(Wb[i], DP, DP) for i in range(DEPTH)]
    ba2 = [_pad2(ba[i][None, :], 1, DP) for i in range(DEPTH)]
    bb2 = [_pad2(bb[i][None, :], 1, DP) for i in range(DEPTH)]

    src = jnp.concatenate([edge_index_r[0], edge_index_p[0]]).astype(jnp.int32)
    dst = jnp.concatenate([edge_index_r[1], edge_index_p[1]]).astype(jnp.int32)
    seg3 = jnp.concatenate([seg_r, seg_p]).astype(jnp.int32).reshape(
        2 * N // 512, 1, 512)

    nf = jnp.concatenate([node_feats_r, node_feats_p]).astype(f32)
    ef = jnp.concatenate([edge_feats_r, edge_feats_p]).astype(f32)

    h = _linear(nf, wn, bn2, True, 512)
    e = _linear(ef, we, be2, False, 2048)

    for i in range(DEPTH - 1):
        agg = _edge_agg(h, e, src, dst)
        h = _mlp(h, agg, wa[i], ba2[i], wb[i], bb2[i], True)

    agg = _edge_agg(h, e, src, dst)
    diff, react, prod = _final(h, agg, wa[2], ba2[2], wb[2], bb2[2], seg3)
    return (diff[:, :D], react[:, :D], prod[:, :D])


# speedup vs baseline: 1.0604x; 1.0034x over previous
import functools

import jax
import jax.numpy as jnp
from jax import lax
from jax.experimental import pallas as pl
from jax.experimental.pallas import tpu as pltpu
from jax.experimental.pallas import tpu_sc as plsc

N = 4096
E = 16384
B = 16
D = 300
DP = 384
DEPTH = 3
NC = 2
NS = 16
CHUNK = 1024
NCH = E // CHUNK
PP = 16
RPT = N // NS
LANES = 16


def _pad2(w, rows, cols):
    return jnp.zeros((rows, cols), w.dtype).at[: w.shape[0], : w.shape[1]].set(w)


def _linear(x, w, b, relu, block_rows):
    m, k = x.shape
    dp = w.shape[1]

    def body(x_ref, w_ref, b_ref, o_ref):
        y = jnp.dot(x_ref[...], w_ref[...],
                    preferred_element_type=jnp.float32) + b_ref[...]
        o_ref[...] = jnp.maximum(y, 0.0) if relu else y

    return pl.pallas_call(
        body,
        grid=(m // block_rows,),
        in_specs=[
            pl.BlockSpec((block_rows, k), lambda i: (i, 0)),
            pl.BlockSpec((k, dp), lambda i: (0, 0)),
            pl.BlockSpec((1, dp), lambda i: (0, 0)),
        ],
        out_specs=pl.BlockSpec((block_rows, dp), lambda i: (i, 0)),
        out_shape=jax.ShapeDtypeStruct((m, dp), jnp.float32),
    )(x, w, b)


def _mlp(h, agg, wa, ba, wb, bb, relu, block_rows=512):
    rows = h.shape[0]

    def body(h_ref, a_ref, wa_ref, ba_ref, wb_ref, bb_ref, o_ref):
        z = h_ref[...] + a_ref[...]
        t = jnp.maximum(
            jnp.dot(z, wa_ref[...], preferred_element_type=jnp.float32)
            + ba_ref[...], 0.0)
        y = jnp.dot(t, wb_ref[...],
                    preferred_element_type=jnp.float32) + bb_ref[...]
        o_ref[...] = jnp.maximum(y, 0.0) if relu else y

    return pl.pallas_call(
        body,
        grid=(rows // block_rows,),
        in_specs=[
            pl.BlockSpec((block_rows, DP), lambda i: (i, 0)),
            pl.BlockSpec((block_rows, DP), lambda i: (i, 0)),
            pl.BlockSpec((DP, DP), lambda i: (0, 0)),
            pl.BlockSpec((1, DP), lambda i: (0, 0)),
            pl.BlockSpec((DP, DP), lambda i: (0, 0)),
            pl.BlockSpec((1, DP), lambda i: (0, 0)),
        ],
        out_specs=pl.BlockSpec((block_rows, DP), lambda i: (i, 0)),
        out_shape=jax.ShapeDtypeStruct((rows, DP), jnp.float32),
    )(h, agg, wa, ba, wb, bb)


def _final(h, agg, wa, ba, wb, bb, seg3, block_rows=512):
    nblk = 2 * N // block_rows
    half = N // block_rows

    def body(h_ref, a_ref, wa_ref, ba_ref, wb_ref, bb_ref, sg,
             diff_ref, re_ref, pr_ref):
        j = pl.program_id(0)
        z = h_ref[...] + a_ref[...]
        t = jnp.maximum(
            jnp.dot(z, wa_ref[...], preferred_element_type=jnp.float32)
            + ba_ref[...], 0.0)
        o = jnp.dot(t, wb_ref[...],
                    preferred_element_type=jnp.float32) + bb_ref[...]
        iota = lax.broadcasted_iota(jnp.int32, (B, block_rows), 0)
        ct = jnp.dot((sg[0] == iota).astype(jnp.float32), o,
                     preferred_element_type=jnp.float32)

        @pl.when(j == 0)
        def _():
            re_ref[...] = ct

        @pl.when((j > 0) & (j < half))
        def _():
            re_ref[...] += ct

        @pl.when(j == half)
        def _():
            pr_ref[...] = ct

        @pl.when(j > half)
        def _():
            pr_ref[...] += ct

        @pl.when(j == nblk - 1)
        def _():
            diff_ref[...] = re_ref[...] - pr_ref[...]

    row_spec = pl.BlockSpec((block_rows, DP), lambda i: (i, 0))
    full = pl.BlockSpec((DP, DP), lambda i: (0, 0))
    bias = pl.BlockSpec((1, DP), lambda i: (0, 0))
    seg_spec = pl.BlockSpec((1, 1, block_rows), lambda i: (i, 0, 0))
    out_spec = pl.BlockSpec((B, DP), lambda i: (0, 0))
    return pl.pallas_call(
        body,
        grid=(nblk,),
        in_specs=[row_spec, row_spec, full, bias, full, bias, seg_spec],
        out_specs=(out_spec, out_spec, out_spec),
        out_shape=(jax.ShapeDtypeStruct((B, DP), jnp.float32),
                   jax.ShapeDtypeStruct((B, DP), jnp.float32),
                   jax.ShapeDtypeStruct((B, DP), jnp.float32)),
    )(h, agg, wa, ba, wb, bb, seg3)


def _edge_agg(h, e, src, dst):
    mesh = plsc.VectorSubcoreMesh(core_axis_name="c", subcore_axis_name="s",
                                  num_cores=NC, num_subcores=NS)

    @functools.partial(
        pl.kernel,
        out_type=jax.ShapeDtypeStruct((2 * N, DP), jnp.float32),
        mesh=mesh,
        compiler_params=pltpu.CompilerParams(needs_layout_passes=False),
        scratch_types=[
            pltpu.VMEM((CHUNK,), jnp.int32),
            pltpu.VMEM((CHUNK,), jnp.int32),
            pltpu.VMEM((CHUNK + PP,), jnp.int32),
            pltpu.VMEM((CHUNK + PP,), jnp.int32),
            pltpu.VMEM((CHUNK + PP,), jnp.int32),
            pltpu.VMEM((2 * PP, DP), jnp.float32),
            pltpu.VMEM((2 * PP, DP), jnp.float32),
            pltpu.VMEM((RPT, DP), jnp.float32),
            pltpu.SemaphoreType.DMA,
            pltpu.SemaphoreType.DMA,
            pltpu.SemaphoreType.DMA,
            pltpu.SemaphoreType.DMA,
            pltpu.SemaphoreType.DMA,
            pltpu.SemaphoreType.DMA,
        ],
    )
    def k(hh, ee, ss, dd, out,
          dbuf, sbuf, plist, slist, rlist, hbuf, ebuf, acc,
          semd, semsrc, semh0, seme0, semh1, seme1):
        cid = lax.axis_index("c")
        sid = lax.axis_index("s")
        row0 = sid * RPT
        ebase = cid * E
        hoff0 = cid * N
        zero16 = jnp.zeros((LANES,), jnp.float32)
        zero16i = jnp.zeros((LANES,), jnp.int32)

        def zacc(r, carry):
            for j in range(DP // LANES):
                acc[r, pl.ds(j * LANES, LANES)] = zero16
            return carry

        lax.fori_loop(0, RPT, zacc, 0)

        def zidx(v, carry):
            plist[pl.ds(v * LANES, LANES)] = zero16i
            slist[pl.ds(v * LANES, LANES)] = zero16i
            return carry

        lax.fori_loop(0, (CHUNK + PP) // LANES, zidx, 0)

        pltpu.async_copy(dd.at[pl.ds(ebase, CHUNK)], dbuf, semd)
        pltpu.async_copy(ss.at[pl.ds(ebase, CHUNK)], sbuf, semsrc)

        def issue(off, half, sh, se):
            pltpu.async_copy(hh.at[slist.at[pl.ds(off, PP)]],
                             hbuf.at[pl.ds(half * PP, PP)], sh)
            pltpu.async_copy(ee.at[plist.at[pl.ds(off, PP)]],
                             ebuf.at[pl.ds(half * PP, PP)], se)

        def drain(off, half, sh, se):
            pltpu.make_async_copy(hh.at[slist.at[pl.ds(off, PP)]],
                                  hbuf.at[pl.ds(half * PP, PP)], sh).wait()
            pltpu.make_async_copy(ee.at[plist.at[pl.ds(off, PP)]],
                                  ebuf.at[pl.ds(half * PP, PP)], se).wait()

        def chunk_body(kk, carry):
            base = ebase + kk * CHUNK
            pltpu.make_async_copy(dd.at[pl.ds(base, CHUNK)], dbuf,
                                  semd).wait()
            pltpu.make_async_copy(ss.at[pl.ds(base, CHUNK)], sbuf,
                                  semsrc).wait()

            def scan(v, ncur):
                sl = pl.ds(v * LANES, LANES)
                ids = dbuf[sl]
                srcs = sbuf[sl]
                mask = (ids >= row0) & (ids < row0 + RPT)
                iota = lax.broadcasted_iota(jnp.int32, (LANES,), 0)
                pos = iota + (base + v * LANES)
                mi = mask.astype(jnp.int32)
                pref = mi
                for kshift in (1, 2, 4, 8):
                    shifted = jnp.take_along_axis(
                        pref, jnp.maximum(iota - kshift, 0), axis=0)
                    pref = pref + jnp.where(iota >= kshift, shifted, 0)
                idxv = pref - mi + ncur
                plsc.store_scatter(plist, [idxv], pos, mask=mask)
                plsc.store_scatter(slist, [idxv], srcs + hoff0, mask=mask)
                plsc.store_scatter(rlist, [idxv], ids - row0, mask=mask)
                cnt = plsc.all_reduce_population_count(mask)
                return ncur + cnt[0]

            n = lax.fori_loop(0, CHUNK // LANES, scan, jnp.int32(0))

            @pl.when(kk + 1 < NCH)
            def _():
                pltpu.async_copy(dd.at[pl.ds(base + CHUNK, CHUNK)],
                                 dbuf, semd)
                pltpu.async_copy(ss.at[pl.ds(base + CHUNK, CHUNK)],
                                 sbuf, semsrc)

            @pl.when(n > 0)
            def _():
                issue(0, 0, semh0, seme0)

            def gb_body(b, carry2):
                off = b * PP
                par = b % 2

                @pl.when(par == 0)
                def _():
                    drain(off, 0, semh0, seme0)

                @pl.when(par == 1)
                def _():
                    drain(off, 1, semh1, seme1)

                @pl.when(off + PP < n)
                def _():
                    @pl.when(par == 0)
                    def _():
                        issue(off + PP, 1, semh1, seme1)

                    @pl.when(par == 1)
                    def _():
                        issue(off + PP, 0, semh0, seme0)

                hoff = par * PP

                def rowacc(r, carry3):
                    lrow = rlist[pl.ds(off + r, LANES)][0]
                    hrow = hoff + r
                    for j in range(DP // LANES):
                        sl2 = pl.ds(j * LANES, LANES)
                        plsc.addupdate(acc.at[lrow, sl2], jnp.maximum(
                            hbuf[hrow, sl2] + ebuf[hrow, sl2], 0.0))
                    return carry3

                lax.fori_loop(0, jnp.minimum(PP, n - off), rowacc, 0)
                return carry2

            lax.fori_loop(0, (n + PP - 1) // PP, gb_body, 0)
            return carry

        lax.fori_loop(0, NCH, chunk_body, 0)
        pltpu.sync_copy(acc, out.at[pl.ds(hoff0 + row0, RPT)])

    return k(h, e, src, dst)


def kernel(node_feats_r, edge_feats_r, node_feats_p, edge_feats_p,
           Wn, bn, We, be, Wa, ba, Wb, bb,
           edge_index_r, seg_r, edge_index_p, seg_p):
    f32 = jnp.float32
    wn = _pad2(Wn, 64, DP)
    we = _pad2(We, 8, DP)
    bn2 = _pad2(bn[None, :], 1, DP)
    be2 = _pad2(be[None, :], 1, DP)
    wa = [_pad2(Wa[i], DP, DP) for i in range(DEPTH)]
    wb = [_pad2(Wb[i], DP, DP) for i in range(DEPTH)]
    ba2 = [_pad2(ba[i][None, :], 1, DP) for i in range(DEPTH)]
    bb2 = [_pad2(bb[i][None, :], 1, DP) for i in range(DEPTH)]

    src = jnp.concatenate([edge_index_r[0], edge_index_p[0]]).astype(jnp.int32)
    dst = jnp.concatenate([edge_index_r[1], edge_index_p[1]]).astype(jnp.int32)
    seg3 = jnp.concatenate([seg_r, seg_p]).astype(jnp.int32).reshape(
        2 * N // 512, 1, 512)

    nf = jnp.concatenate([node_feats_r, node_feats_p]).astype(f32)
    ef = jnp.concatenate([edge_feats_r, edge_feats_p]).astype(f32)

    h = _linear(nf, wn, bn2, True, 512)
    e = _linear(ef, we, be2, False, 2048)

    for i in range(DEPTH - 1):
        agg = _edge_agg(h, e, src, dst)
        h = _mlp(h, agg, wa[i], ba2[i], wb[i], bb2[i], True)

    agg = _edge_agg(h, e, src, dst)
    diff, react, prod = _final(h, agg, wa[2], ba2[2], wb[2], bb2[2], seg3)
    return (diff[:, :D], react[:, :D], prod[:, :D])
